# Initial kernel scaffold; baseline (speedup 1.0000x reference)
import jax
import jax.numpy as jnp
from jax.experimental import pallas as pl


def kernel(edge_index, edge_type, W1, root1, bias1, W2, root2, bias2):
    raise NotImplementedError("write your pallas kernel here")


# speedup vs baseline: 17.3541x; 17.3541x over previous
import functools

import jax
import jax.numpy as jnp
from jax import lax
from jax.experimental import pallas as pl
from jax.experimental.pallas import tpu as pltpu
from jax.experimental.pallas import tpu_sc as plsc

NC = 2
NS = 16
CHUNK = 128


def _mesh():
    return plsc.VectorSubcoreMesh(core_axis_name="c", subcore_axis_name="s")


def _make_scales_kernel(Ep, SEGN, R):
    chunks_all = Ep // (NS * CHUNK)
    chunks_half = Ep // (NC * NS * CHUNK)
    per_tile = SEGN // NS
    BUFZ = per_tile // 8

    @functools.partial(
        pl.kernel,
        out_type=jax.ShapeDtypeStruct((Ep,), jnp.float32),
        mesh=_mesh(),
        scratch_types=[
            pltpu.VMEM((CHUNK,), jnp.int32),
            pltpu.VMEM((CHUNK,), jnp.int32),
            pltpu.VMEM((CHUNK,), jnp.int32),
            pltpu.VMEM((CHUNK,), jnp.float32),
            pltpu.VMEM((BUFZ,), jnp.float32),
            pltpu.VMEM_SHARED((SEGN,), jnp.float32),
        ],
    )
    def k(dst_hbm, rel_hbm, zc_hbm, ones_hbm, scales_hbm,
          dstv, relv, segv, fv, zbuf, cnt_sp):
        c = lax.axis_index("c")
        s = lax.axis_index("s")
        w = c * NS + s

        pltpu.sync_copy(zc_hbm.at[pl.ds(s * per_tile, per_tile)],
                        cnt_sp.at[pl.ds(s * per_tile, per_tile)])
        pltpu.sync_copy(ones_hbm, fv)
        plsc.subcore_barrier()

        def p1(g, carry):
            base = (s * chunks_all + g) * CHUNK
            pltpu.sync_copy(dst_hbm.at[pl.ds(base, CHUNK)], dstv)
            pltpu.sync_copy(rel_hbm.at[pl.ds(base, CHUNK)], relv)
            for i in range(CHUNK // 16):
                sl = pl.ds(i * 16, 16)
                segv[sl] = dstv[sl] * R + relv[sl]
            pltpu.sync_copy(fv, cnt_sp.at[segv], add=True)
            return carry
        lax.fori_loop(0, chunks_all, p1, 0)
        plsc.subcore_barrier()

        for j in range(8):
            off = s * per_tile + j * BUFZ
            pltpu.sync_copy(cnt_sp.at[pl.ds(off, BUFZ)], zbuf)

            def inv(jj, carry):
                sl = pl.ds(jj * 16, 16)
                zbuf[sl] = 1.0 / jnp.maximum(zbuf[sl], 1.0)
                return carry
            lax.fori_loop(0, BUFZ // 16, inv, 0)
            pltpu.sync_copy(zbuf, cnt_sp.at[pl.ds(off, BUFZ)])
        plsc.subcore_barrier()

        def p3(g, carry):
            base = (w * chunks_half + g) * CHUNK
            pltpu.sync_copy(dst_hbm.at[pl.ds(base, CHUNK)], dstv)
            pltpu.sync_copy(rel_hbm.at[pl.ds(base, CHUNK)], relv)
            for i in range(CHUNK // 16):
                sl = pl.ds(i * 16, 16)
                segv[sl] = dstv[sl] * R + relv[sl]
            pltpu.sync_copy(cnt_sp.at[segv], fv)
            pltpu.sync_copy(fv, scales_hbm.at[pl.ds(base, CHUNK)])
            return carry
        lax.fori_loop(0, chunks_half, p3, 0)

    return k


def _make_agg_kernel(Ep, T, M, AGGR, H):
    chunks = Ep // (NC * NS * CHUNK)
    rows_pt = AGGR // NS

    @functools.partial(
        pl.kernel,
        out_type=(jax.ShapeDtypeStruct((AGGR, H), jnp.float32),
                  jax.ShapeDtypeStruct((AGGR, H), jnp.float32)),
        mesh=_mesh(),
        compiler_params=pltpu.CompilerParams(use_tc_tiling_on_sc=False),
        scratch_types=[
            pltpu.VMEM((CHUNK,), jnp.int32),
            pltpu.VMEM((CHUNK,), jnp.int32),
            pltpu.VMEM((CHUNK,), jnp.int32),
            pltpu.VMEM((CHUNK,), jnp.int32),
            pltpu.VMEM((CHUNK,), jnp.float32),
            pltpu.VMEM((CHUNK, H), jnp.float32),
            pltpu.VMEM_SHARED((AGGR, H), jnp.float32),
        ],
    )
    def k(ia_hbm, ib_hbm, dst_hbm, sc_hbm, tab_hbm, zr_hbm, o0_hbm, o1_hbm,
          iav, ibv, dstv, gidxv, sv, rows_v, agg_sp):
        c = lax.axis_index("c")
        s = lax.axis_index("s")
        w = c * NS + s
        r0 = s * rows_pt

        pltpu.sync_copy(zr_hbm.at[pl.ds(r0, rows_pt), :],
                        agg_sp.at[pl.ds(r0, rows_pt), :])
        plsc.subcore_barrier()

        def body(g, carry):
            base = (w * chunks + g) * CHUNK
            pltpu.sync_copy(ia_hbm.at[pl.ds(base, CHUNK)], iav)
            pltpu.sync_copy(ib_hbm.at[pl.ds(base, CHUNK)], ibv)
            pltpu.sync_copy(dst_hbm.at[pl.ds(base, CHUNK)], dstv)
            pltpu.sync_copy(sc_hbm.at[pl.ds(base, CHUNK)], sv)
            for i in range(CHUNK // 16):
                sl = pl.ds(i * 16, 16)
                gidxv[sl] = iav[sl] * M + ibv[sl]
            pltpu.sync_copy(tab_hbm.at[gidxv], rows_v)
            for g8 in range(CHUNK // 16):
                s16 = sv[pl.ds(g8 * 16, 16)]
                for ei in range(16):
                    e = g8 * 16 + ei
                    rows_v[e, :] = rows_v[e, :] * s16[ei]
            pltpu.sync_copy(rows_v, agg_sp.at[dstv], add=True)
            return carry
        lax.fori_loop(0, chunks, body, 0)
        plsc.subcore_barrier()

        @pl.when(c == 0)
        def _():
            pltpu.sync_copy(agg_sp.at[pl.ds(r0, rows_pt), :],
                            o0_hbm.at[pl.ds(r0, rows_pt), :])

        @pl.when(c == 1)
        def _():
            pltpu.sync_copy(agg_sp.at[pl.ds(r0, rows_pt), :],
                            o1_hbm.at[pl.ds(r0, rows_pt), :])

    return k


def _tc_mid(p0, p1, root1, bias1, W2t, root2, bias2, N, H, RL):
    Bn = 5000
    grid = (N // Bn,)

    def body(p0_ref, p1_ref, r1_ref, b1_ref, w2_ref, r2_ref, b2_ref,
             xw_ref, ob_ref):
        x = p0_ref[...] + p1_ref[...] + r1_ref[...] + b1_ref[...]
        x = jnp.maximum(x, 0.0)
        xw_ref[...] = jnp.dot(x, w2_ref[...], preferred_element_type=jnp.float32)
        ob_ref[...] = (jnp.dot(x, r2_ref[...], preferred_element_type=jnp.float32)
                       + b2_ref[...])

    row_spec = pl.BlockSpec((Bn, H), lambda i: (i, 0))
    full = lambda shp: pl.BlockSpec(shp, lambda i: (0, 0))
    return pl.pallas_call(
        body,
        grid=grid,
        in_specs=[row_spec, row_spec, row_spec, full((1, H)),
                  full((H, RL)), full((H, H)), full((1, H))],
        out_specs=[pl.BlockSpec((Bn, RL), lambda i: (i, 0)), row_spec],
        out_shape=(jax.ShapeDtypeStruct((N, RL), jnp.float32),
                   jax.ShapeDtypeStruct((N, H), jnp.float32)),
    )(p0, p1, root1, bias1, W2t, root2, bias2)


def _tc_final(q0, q1, ob, N, H):
    Bn = 5000
    grid = (N // Bn,)

    def body(q0_ref, q1_ref, ob_ref, o_ref):
        o_ref[...] = jax.nn.sigmoid(q0_ref[...] + q1_ref[...] + ob_ref[...])

    row_spec = pl.BlockSpec((Bn, H), lambda i: (i, 0))
    return pl.pallas_call(
        body,
        grid=grid,
        in_specs=[row_spec, row_spec, row_spec],
        out_specs=row_spec,
        out_shape=jax.ShapeDtypeStruct((N, H), jnp.float32),
    )(q0, q1, ob)


def kernel(edge_index, edge_type, W1, root1, bias1, W2, root2, bias2):
    R, N, H = W1.shape
    L = W2.shape[2]
    E = edge_index.shape[1]

    EW = NC * NS * CHUNK
    Ep = ((E + EW - 1) // EW) * EW
    pad = Ep - E
    src = jnp.concatenate(
        [edge_index[0].astype(jnp.int32), jnp.zeros((pad,), jnp.int32)])
    dst = jnp.concatenate(
        [edge_index[1].astype(jnp.int32),
         N + (jnp.arange(pad, dtype=jnp.int32) % 64)])
    rel = jnp.concatenate(
        [edge_type.astype(jnp.int32), jnp.zeros((pad,), jnp.int32)])

    AGGR = ((N + 96) // NS + 7) // 8 * 8 * NS
    SEGN = ((N + 64) * R + NS * 64 - 1) // (NS * 64) * (NS * 64)

    zc = jnp.zeros((SEGN,), jnp.float32)
    ones = jnp.ones((CHUNK,), jnp.float32)
    zr = jnp.zeros((AGGR, H), jnp.float32)

    scales = _make_scales_kernel(Ep, SEGN, R)(dst, rel, zc, ones)

    W1f = W1.reshape(R * N, H)
    p0, p1 = _make_agg_kernel(Ep, R * N, N, AGGR, H)(
        rel, src, dst, scales, W1f, zr)

    W2t = jnp.transpose(W2, (1, 0, 2)).reshape(H, R * L)
    root1b = root1
    xw, ob = _tc_mid(p0[:N], p1[:N], root1b, bias1.reshape(1, H),
                     W2t, root2, bias2.reshape(1, L), N, H, R * L)

    q0, q1 = _make_agg_kernel(Ep, N * R, R, AGGR, L)(
        src, rel, dst, scales, xw.reshape(N * R, L), zr)

    return _tc_final(q0[:N], q1[:N], ob, N, L)


# speedup vs baseline: 41.0584x; 2.3659x over previous
import functools

import jax
import jax.numpy as jnp
from jax import lax
from jax.experimental import pallas as pl
from jax.experimental.pallas import tpu as pltpu
from jax.experimental.pallas import tpu_sc as plsc

NC = 2
NS = 16
CHUNK = 128
ISRC, IDST, IREL = 0, 1, 2


def _mesh():
    return plsc.VectorSubcoreMesh(core_axis_name="c", subcore_axis_name="s")


def _seg_from(eb, segv, R):
    for i in range(CHUNK // 16):
        sl = pl.ds(i * 16, 16)
        segv[sl] = eb[IDST, sl] * R + eb[IREL, sl]


def _make_scales_kernel(Ep, SEGN, R):
    chunks_all = Ep // (NS * CHUNK)
    chunks_half = Ep // (NC * NS * CHUNK)
    per_tile = SEGN // NS
    BUFZ = per_tile // 8

    @functools.partial(
        pl.kernel,
        out_type=jax.ShapeDtypeStruct((Ep,), jnp.float32),
        mesh=_mesh(),
        compiler_params=pltpu.CompilerParams(use_tc_tiling_on_sc=False),
        scratch_types=[
            pltpu.VMEM((3, CHUNK), jnp.int32),
            pltpu.VMEM((3, CHUNK), jnp.int32),
            pltpu.VMEM((CHUNK,), jnp.int32),
            pltpu.VMEM((CHUNK,), jnp.int32),
            pltpu.VMEM((CHUNK,), jnp.float32),
            pltpu.VMEM((CHUNK,), jnp.float32),
            pltpu.VMEM((CHUNK,), jnp.float32),
            pltpu.VMEM((BUFZ,), jnp.float32),
            pltpu.VMEM_SHARED((SEGN,), jnp.float32),
            pltpu.SemaphoreType.DMA,
            pltpu.SemaphoreType.DMA,
            pltpu.SemaphoreType.DMA,
            pltpu.SemaphoreType.DMA,
        ],
    )
    def k(eb_hbm, zc_hbm, ones_hbm, scales_hbm,
          eb0, eb1, segv0, segv1, sv0, sv1, fv, zbuf, cnt_sp,
          lin0, lin1, st0, st1):
        c = lax.axis_index("c")
        s = lax.axis_index("s")
        w = c * NS + s
        eb = (eb0, eb1)
        segv = (segv0, segv1)
        sv = (sv0, sv1)
        lin = (lin0, lin1)
        st = (st0, st1)

        pltpu.sync_copy(zc_hbm.at[pl.ds(s * per_tile, per_tile)],
                        cnt_sp.at[pl.ds(s * per_tile, per_tile)])
        pltpu.sync_copy(ones_hbm, fv)
        plsc.subcore_barrier()

        base1 = s * chunks_all
        pltpu.async_copy(eb_hbm.at[base1], eb[0], lin[0])

        def p1(g, carry):
            for b in range(2):
                p, n = b, 1 - b
                cc = 2 * g + b
                @pl.when(cc + 1 < chunks_all)
                def _():
                    pltpu.async_copy(eb_hbm.at[base1 + cc + 1], eb[n], lin[n])
                pltpu.make_async_copy(eb_hbm.at[base1 + cc], eb[p],
                                      lin[p]).wait()
                @pl.when(g >= 1)
                def _():
                    pltpu.make_async_copy(
                        fv, cnt_sp.at[segv[p]], st[p]).wait()
                _seg_from(eb[p], segv[p], R)
                pltpu.async_copy(fv, cnt_sp.at[segv[p]], st[p], add=True)
            return carry
        lax.fori_loop(0, chunks_all // 2, p1, 0)
        pltpu.make_async_copy(fv, cnt_sp.at[segv[0]], st[0]).wait()
        pltpu.make_async_copy(fv, cnt_sp.at[segv[1]], st[1]).wait()
        plsc.subcore_barrier()

        for j in range(8):
            off = s * per_tile + j * BUFZ
            pltpu.sync_copy(cnt_sp.at[pl.ds(off, BUFZ)], zbuf)

            def inv(jj, carry):
                sl = pl.ds(jj * 16, 16)
                zbuf[sl] = 1.0 / jnp.maximum(zbuf[sl], 1.0)
                return carry
            lax.fori_loop(0, BUFZ // 16, inv, 0)
            pltpu.sync_copy(zbuf, cnt_sp.at[pl.ds(off, BUFZ)])
        plsc.subcore_barrier()

        base3 = w * chunks_half
        pltpu.async_copy(eb_hbm.at[base3], eb[0], lin[0])

        def p3(g, carry):
            for b in range(2):
                p, n = b, 1 - b
                cc = 2 * g + b
                @pl.when(cc + 1 < chunks_half)
                def _():
                    pltpu.async_copy(eb_hbm.at[base3 + cc + 1], eb[n], lin[n])
                pltpu.make_async_copy(eb_hbm.at[base3 + cc], eb[p],
                                      lin[p]).wait()
                @pl.when(g >= 1)
                def _():
                    pltpu.make_async_copy(
                        sv[p],
                        scales_hbm.at[pl.ds((base3 + cc - 2) * CHUNK, CHUNK)],
                        st[p]).wait()
                _seg_from(eb[p], segv[p], R)
                pltpu.sync_copy(cnt_sp.at[segv[p]], sv[p])
                pltpu.async_copy(
                    sv[p],
                    scales_hbm.at[pl.ds((base3 + cc) * CHUNK, CHUNK)],
                    st[p])
            return carry
        lax.fori_loop(0, chunks_half // 2, p3, 0)
        last = base3 + chunks_half
        pltpu.make_async_copy(
            sv[0], scales_hbm.at[pl.ds((last - 2) * CHUNK, CHUNK)],
            st[0]).wait()
        pltpu.make_async_copy(
            sv[1], scales_hbm.at[pl.ds((last - 1) * CHUNK, CHUNK)],
            st[1]).wait()

    return k


def _make_agg_kernel(Ep, T, M, IA, IB, AGGR, H):
    chunks = Ep // (NC * NS * CHUNK)
    rows_pt = AGGR // NS

    @functools.partial(
        pl.kernel,
        out_type=(jax.ShapeDtypeStruct((AGGR, H), jnp.float32),
                  jax.ShapeDtypeStruct((AGGR, H), jnp.float32)),
        mesh=_mesh(),
        compiler_params=pltpu.CompilerParams(use_tc_tiling_on_sc=False),
        scratch_types=[
            pltpu.VMEM((3, CHUNK), jnp.int32),
            pltpu.VMEM((3, CHUNK), jnp.int32),
            pltpu.VMEM((CHUNK,), jnp.int32),
            pltpu.VMEM((CHUNK,), jnp.int32),
            pltpu.VMEM((CHUNK,), jnp.float32),
            pltpu.VMEM((CHUNK,), jnp.float32),
            pltpu.VMEM((CHUNK,), jnp.int32),
            pltpu.VMEM((CHUNK,), jnp.int32),
            pltpu.VMEM((CHUNK, H), jnp.float32),
            pltpu.VMEM((CHUNK, H), jnp.float32),
            pltpu.VMEM_SHARED((AGGR, H), jnp.float32),
            pltpu.SemaphoreType.DMA,
            pltpu.SemaphoreType.DMA,
            pltpu.SemaphoreType.DMA,
            pltpu.SemaphoreType.DMA,
            pltpu.SemaphoreType.DMA,
            pltpu.SemaphoreType.DMA,
        ],
    )
    def k(eb_hbm, sc_hbm, tab_hbm, zr_hbm, o0_hbm, o1_hbm,
          eb0, eb1, gidx0, gidx1, sv0, sv1, dstv0, dstv1, rows0, rows1,
          agg_sp, lin0, lin1, gs0, gs1, ss0, ss1):
        c = lax.axis_index("c")
        s = lax.axis_index("s")
        w = c * NS + s
        r0 = s * rows_pt
        eb = (eb0, eb1)
        gidx = (gidx0, gidx1)
        sv = (sv0, sv1)
        dstv = (dstv0, dstv1)
        rows = (rows0, rows1)
        lin = (lin0, lin1)
        gsem = (gs0, gs1)
        ssem = (ss0, ss1)

        pltpu.sync_copy(zr_hbm.at[pl.ds(r0, rows_pt), :],
                        agg_sp.at[pl.ds(r0, rows_pt), :])
        plsc.subcore_barrier()

        base = w * chunks

        def fire_lin(cc, b):
            pltpu.async_copy(eb_hbm.at[base + cc], eb[b], lin[b])
            pltpu.async_copy(sc_hbm.at[pl.ds((base + cc) * CHUNK, CHUNK)],
                             sv[b], lin[b])

        def wait_lin(cc, b):
            pltpu.make_async_copy(eb_hbm.at[base + cc], eb[b], lin[b]).wait()
            pltpu.make_async_copy(
                sc_hbm.at[pl.ds((base + cc) * CHUNK, CHUNK)], sv[b],
                lin[b]).wait()

        def fire_gather(b):
            pltpu.async_copy(tab_hbm.at[gidx[b]], rows[b], gsem[b])

        fire_lin(0, 0)
        wait_lin(0, 0)
        for i in range(CHUNK // 16):
            sl = pl.ds(i * 16, 16)
            gidx[0][sl] = eb[0][IA, sl] * M + eb[0][IB, sl]
        fire_gather(0)

        def body(g, carry):
            for b in range(2):
                p, n = b, 1 - b
                cc = 2 * g + b
                @pl.when(cc + 1 < chunks)
                def _():
                    fire_lin(cc + 1, n)
                pltpu.make_async_copy(tab_hbm.at[gidx[p]], rows[p],
                                      gsem[p]).wait()
                for g8 in range(CHUNK // 16):
                    sl = pl.ds(g8 * 16, 16)
                    dstv[p][sl] = eb[p][IDST, sl]
                    s16 = sv[p][sl]
                    for ei in range(16):
                        e = g8 * 16 + ei
                        rows[p][e, :] = rows[p][e, :] * s16[ei]
                pltpu.async_copy(rows[p], agg_sp.at[dstv[p]], ssem[p],
                                 add=True)
                @pl.when(cc + 1 < chunks)
                def _():
                    wait_lin(cc + 1, n)
                    @pl.when(cc >= 1)
                    def _():
                        pltpu.make_async_copy(
                            rows[n], agg_sp.at[dstv[n]], ssem[n]).wait()
                    for i in range(CHUNK // 16):
                        sl = pl.ds(i * 16, 16)
                        gidx[n][sl] = (eb[n][IA, sl] * M + eb[n][IB, sl])
                    fire_gather(n)
            return carry
        lax.fori_loop(0, chunks // 2, body, 0)
        pltpu.make_async_copy(rows[0], agg_sp.at[dstv[0]], ssem[0]).wait()
        pltpu.make_async_copy(rows[1], agg_sp.at[dstv[1]], ssem[1]).wait()
        plsc.subcore_barrier()

        @pl.when(c == 0)
        def _():
            pltpu.sync_copy(agg_sp.at[pl.ds(r0, rows_pt), :],
                            o0_hbm.at[pl.ds(r0, rows_pt), :])

        @pl.when(c == 1)
        def _():
            pltpu.sync_copy(agg_sp.at[pl.ds(r0, rows_pt), :],
                            o1_hbm.at[pl.ds(r0, rows_pt), :])

    return k


def _tc_mid(p0, p1, root1, bias1, W2t, root2, bias2, N, H, RL):
    Bn = 5000
    grid = (N // Bn,)

    def body(p0_ref, p1_ref, r1_ref, b1_ref, w2_ref, r2_ref, b2_ref,
             xw_ref, ob_ref):
        x = p0_ref[...] + p1_ref[...] + r1_ref[...] + b1_ref[...]
        x = jnp.maximum(x, 0.0)
        xw_ref[...] = jnp.dot(x, w2_ref[...], preferred_element_type=jnp.float32)
        ob_ref[...] = (jnp.dot(x, r2_ref[...], preferred_element_type=jnp.float32)
                       + b2_ref[...])

    row_spec = pl.BlockSpec((Bn, H), lambda i: (i, 0))
    full = lambda shp: pl.BlockSpec(shp, lambda i: (0, 0))
    return pl.pallas_call(
        body,
        grid=grid,
        in_specs=[row_spec, row_spec, row_spec, full((1, H)),
                  full((H, RL)), full((H, H)), full((1, H))],
        out_specs=[pl.BlockSpec((Bn, RL), lambda i: (i, 0)), row_spec],
        out_shape=(jax.ShapeDtypeStruct((N, RL), jnp.float32),
                   jax.ShapeDtypeStruct((N, H), jnp.float32)),
    )(p0, p1, root1, bias1, W2t, root2, bias2)


def _tc_final(q0, q1, ob, N, H):
    Bn = 5000
    grid = (N // Bn,)

    def body(q0_ref, q1_ref, ob_ref, o_ref):
        o_ref[...] = jax.nn.sigmoid(q0_ref[...] + q1_ref[...] + ob_ref[...])

    row_spec = pl.BlockSpec((Bn, H), lambda i: (i, 0))
    return pl.pallas_call(
        body,
        grid=grid,
        in_specs=[row_spec, row_spec, row_spec],
        out_specs=row_spec,
        out_shape=jax.ShapeDtypeStruct((N, H), jnp.float32),
    )(q0, q1, ob)


def kernel(edge_index, edge_type, W1, root1, bias1, W2, root2, bias2):
    R, N, H = W1.shape
    L = W2.shape[2]
    E = edge_index.shape[1]

    EW = NC * NS * CHUNK * 2
    Ep = ((E + EW - 1) // EW) * EW
    pad = Ep - E
    src = jnp.concatenate(
        [edge_index[0].astype(jnp.int32), jnp.zeros((pad,), jnp.int32)])
    dst = jnp.concatenate(
        [edge_index[1].astype(jnp.int32),
         N + (jnp.arange(pad, dtype=jnp.int32) % 64)])
    rel = jnp.concatenate(
        [edge_type.astype(jnp.int32), jnp.zeros((pad,), jnp.int32)])
    eb = jnp.stack([src.reshape(-1, CHUNK), dst.reshape(-1, CHUNK),
                    rel.reshape(-1, CHUNK)], axis=1)

    AGGR = ((N + 96) // NS + 7) // 8 * 8 * NS
    SEGN = ((N + 64) * R + NS * 64 - 1) // (NS * 64) * (NS * 64)

    zc = jnp.zeros((SEGN,), jnp.float32)
    ones = jnp.ones((CHUNK,), jnp.float32)
    zr = jnp.zeros((AGGR, H), jnp.float32)

    scales = _make_scales_kernel(Ep, SEGN, R)(eb, zc, ones)

    W1f = W1.reshape(R * N, H)
    p0, p1 = _make_agg_kernel(Ep, R * N, N, IREL, ISRC, AGGR, H)(
        eb, scales, W1f, zr)

    W2t = jnp.transpose(W2, (1, 0, 2)).reshape(H, R * L)
    xw, ob = _tc_mid(p0[:N], p1[:N], root1, bias1.reshape(1, H),
                     W2t, root2, bias2.reshape(1, L), N, H, R * L)

    q0, q1 = _make_agg_kernel(Ep, N * R, R, ISRC, IREL, AGGR, L)(
        eb, scales, xw.reshape(N * R, L), zr)

    return _tc_final(q0[:N], q1[:N], ob, N, L)


# speedup vs baseline: 43.9807x; 1.0712x over previous
import functools

import jax
import jax.numpy as jnp
from jax import lax
from jax.experimental import pallas as pl
from jax.experimental.pallas import tpu as pltpu
from jax.experimental.pallas import tpu_sc as plsc

NC = 2
NS = 16
CHUNK = 128
ISRC, IDST, IREL = 0, 1, 2


def _mesh():
    return plsc.VectorSubcoreMesh(core_axis_name="c", subcore_axis_name="s")


def _seg_from(eb, segv, R):
    for i in range(CHUNK // 16):
        sl = pl.ds(i * 16, 16)
        segv[sl] = eb[IDST, sl] * R + eb[IREL, sl]


def _make_scales_kernel(Ep, SEGN, R):
    chunks_all = Ep // (NS * CHUNK)
    chunks_half = Ep // (NC * NS * CHUNK)
    per_tile = SEGN // NS
    BUFZ = per_tile // 8

    @functools.partial(
        pl.kernel,
        out_type=jax.ShapeDtypeStruct((Ep,), jnp.float32),
        mesh=_mesh(),
        compiler_params=pltpu.CompilerParams(use_tc_tiling_on_sc=False),
        scratch_types=[
            pltpu.VMEM((3, CHUNK), jnp.int32),
            pltpu.VMEM((3, CHUNK), jnp.int32),
            pltpu.VMEM((CHUNK,), jnp.int32),
            pltpu.VMEM((CHUNK,), jnp.int32),
            pltpu.VMEM((CHUNK,), jnp.float32),
            pltpu.VMEM((CHUNK,), jnp.float32),
            pltpu.VMEM((CHUNK,), jnp.float32),
            pltpu.VMEM((BUFZ,), jnp.float32),
            pltpu.VMEM_SHARED((SEGN,), jnp.float32),
            pltpu.SemaphoreType.DMA,
            pltpu.SemaphoreType.DMA,
            pltpu.SemaphoreType.DMA,
            pltpu.SemaphoreType.DMA,
        ],
    )
    def k(eb_hbm, zc_hbm, ones_hbm, scales_hbm,
          eb0, eb1, segv0, segv1, sv0, sv1, fv, zbuf, cnt_sp,
          lin0, lin1, st0, st1):
        c = lax.axis_index("c")
        s = lax.axis_index("s")
        w = c * NS + s
        eb = (eb0, eb1)
        segv = (segv0, segv1)
        sv = (sv0, sv1)
        lin = (lin0, lin1)
        st = (st0, st1)

        pltpu.sync_copy(zc_hbm.at[pl.ds(s * per_tile, per_tile)],
                        cnt_sp.at[pl.ds(s * per_tile, per_tile)])
        pltpu.sync_copy(ones_hbm, fv)
        plsc.subcore_barrier()

        base1 = s * chunks_all
        pltpu.async_copy(eb_hbm.at[base1], eb[0], lin[0])

        def p1(g, carry):
            for b in range(2):
                p, n = b, 1 - b
                cc = 2 * g + b
                @pl.when(cc + 1 < chunks_all)
                def _():
                    pltpu.async_copy(eb_hbm.at[base1 + cc + 1], eb[n], lin[n])
                pltpu.make_async_copy(eb_hbm.at[base1 + cc], eb[p],
                                      lin[p]).wait()
                @pl.when(g >= 1)
                def _():
                    pltpu.make_async_copy(
                        fv, cnt_sp.at[segv[p]], st[p]).wait()
                _seg_from(eb[p], segv[p], R)
                pltpu.async_copy(fv, cnt_sp.at[segv[p]], st[p], add=True)
            return carry
        lax.fori_loop(0, chunks_all // 2, p1, 0)
        pltpu.make_async_copy(fv, cnt_sp.at[segv[0]], st[0]).wait()
        pltpu.make_async_copy(fv, cnt_sp.at[segv[1]], st[1]).wait()
        plsc.subcore_barrier()

        for j in range(8):
            off = s * per_tile + j * BUFZ
            pltpu.sync_copy(cnt_sp.at[pl.ds(off, BUFZ)], zbuf)

            def inv(jj, carry):
                sl = pl.ds(jj * 16, 16)
                zbuf[sl] = 1.0 / jnp.maximum(zbuf[sl], 1.0)
                return carry
            lax.fori_loop(0, BUFZ // 16, inv, 0)
            pltpu.sync_copy(zbuf, cnt_sp.at[pl.ds(off, BUFZ)])
        plsc.subcore_barrier()

        base3 = w * chunks_half
        pltpu.async_copy(eb_hbm.at[base3], eb[0], lin[0])

        def p3(g, carry):
            for b in range(2):
                p, n = b, 1 - b
                cc = 2 * g + b
                @pl.when(cc + 1 < chunks_half)
                def _():
                    pltpu.async_copy(eb_hbm.at[base3 + cc + 1], eb[n], lin[n])
                pltpu.make_async_copy(eb_hbm.at[base3 + cc], eb[p],
                                      lin[p]).wait()
                @pl.when(g >= 1)
                def _():
                    pltpu.make_async_copy(
                        sv[p],
                        scales_hbm.at[pl.ds((base3 + cc - 2) * CHUNK, CHUNK)],
                        st[p]).wait()
                _seg_from(eb[p], segv[p], R)
                pltpu.sync_copy(cnt_sp.at[segv[p]], sv[p])
                pltpu.async_copy(
                    sv[p],
                    scales_hbm.at[pl.ds((base3 + cc) * CHUNK, CHUNK)],
                    st[p])
            return carry
        lax.fori_loop(0, chunks_half // 2, p3, 0)
        last = base3 + chunks_half
        pltpu.make_async_copy(
            sv[0], scales_hbm.at[pl.ds((last - 2) * CHUNK, CHUNK)],
            st[0]).wait()
        pltpu.make_async_copy(
            sv[1], scales_hbm.at[pl.ds((last - 1) * CHUNK, CHUNK)],
            st[1]).wait()

    return k


def _make_agg_kernel(Ep, T, M, IA, IB, AGGR, H):
    chunks = Ep // (NC * NS * CHUNK)
    rows_pt = AGGR // NS

    @functools.partial(
        pl.kernel,
        out_type=(jax.ShapeDtypeStruct((AGGR, H), jnp.float32),
                  jax.ShapeDtypeStruct((AGGR, H), jnp.float32)),
        mesh=_mesh(),
        compiler_params=pltpu.CompilerParams(use_tc_tiling_on_sc=False),
        scratch_types=[
            pltpu.VMEM((3, CHUNK), jnp.int32),
            pltpu.VMEM((3, CHUNK), jnp.int32),
            pltpu.VMEM((CHUNK,), jnp.int32),
            pltpu.VMEM((CHUNK,), jnp.int32),
            pltpu.VMEM((CHUNK,), jnp.float32),
            pltpu.VMEM((CHUNK,), jnp.float32),
            pltpu.VMEM((CHUNK,), jnp.int32),
            pltpu.VMEM((CHUNK,), jnp.int32),
            pltpu.VMEM((CHUNK, H), jnp.float32),
            pltpu.VMEM((CHUNK, H), jnp.float32),
            pltpu.VMEM_SHARED((AGGR, H), jnp.float32),
            pltpu.SemaphoreType.DMA,
            pltpu.SemaphoreType.DMA,
            pltpu.SemaphoreType.DMA,
            pltpu.SemaphoreType.DMA,
            pltpu.SemaphoreType.DMA,
            pltpu.SemaphoreType.DMA,
        ],
    )
    def k(eb_hbm, sc_hbm, tab_hbm, zr_hbm, o0_hbm, o1_hbm,
          eb0, eb1, gidx0, gidx1, sv0, sv1, dstv0, dstv1, rows0, rows1,
          agg_sp, lin0, lin1, gs0, gs1, ss0, ss1):
        c = lax.axis_index("c")
        s = lax.axis_index("s")
        w = c * NS + s
        r0 = s * rows_pt
        eb = (eb0, eb1)
        gidx = (gidx0, gidx1)
        sv = (sv0, sv1)
        dstv = (dstv0, dstv1)
        rows = (rows0, rows1)
        lin = (lin0, lin1)
        gsem = (gs0, gs1)
        ssem = (ss0, ss1)

        pltpu.sync_copy(zr_hbm.at[pl.ds(r0, rows_pt), :],
                        agg_sp.at[pl.ds(r0, rows_pt), :])
        plsc.subcore_barrier()

        base = w * chunks

        def fire_lin(cc, b):
            pltpu.async_copy(eb_hbm.at[base + cc], eb[b], lin[b])
            pltpu.async_copy(sc_hbm.at[pl.ds((base + cc) * CHUNK, CHUNK)],
                             sv[b], lin[b])

        def wait_lin(cc, b):
            pltpu.make_async_copy(eb_hbm.at[base + cc], eb[b], lin[b]).wait()
            pltpu.make_async_copy(
                sc_hbm.at[pl.ds((base + cc) * CHUNK, CHUNK)], sv[b],
                lin[b]).wait()

        def fire_gather(b):
            pltpu.async_copy(tab_hbm.at[gidx[b]], rows[b], gsem[b])

        fire_lin(0, 0)
        wait_lin(0, 0)
        for i in range(CHUNK // 16):
            sl = pl.ds(i * 16, 16)
            gidx[0][sl] = eb[0][IA, sl] * M + eb[0][IB, sl]
        fire_gather(0)

        def body(g, carry):
            for b in range(2):
                p, n = b, 1 - b
                cc = 2 * g + b
                @pl.when(cc + 1 < chunks)
                def _():
                    fire_lin(cc + 1, n)
                pltpu.make_async_copy(tab_hbm.at[gidx[p]], rows[p],
                                      gsem[p]).wait()
                for g8 in range(CHUNK // 16):
                    sl = pl.ds(g8 * 16, 16)
                    dstv[p][sl] = eb[p][IDST, sl]
                    s16 = sv[p][sl]
                    for ei in range(16):
                        e = g8 * 16 + ei
                        rows[p][e, :] = rows[p][e, :] * s16[ei]
                pltpu.async_copy(rows[p], agg_sp.at[dstv[p]], ssem[p],
                                 add=True)
                @pl.when(cc + 1 < chunks)
                def _():
                    wait_lin(cc + 1, n)
                    @pl.when(cc >= 1)
                    def _():
                        pltpu.make_async_copy(
                            rows[n], agg_sp.at[dstv[n]], ssem[n]).wait()
                    for i in range(CHUNK // 16):
                        sl = pl.ds(i * 16, 16)
                        gidx[n][sl] = (eb[n][IA, sl] * M + eb[n][IB, sl])
                    fire_gather(n)
            return carry
        lax.fori_loop(0, chunks // 2, body, 0)
        pltpu.make_async_copy(rows[0], agg_sp.at[dstv[0]], ssem[0]).wait()
        pltpu.make_async_copy(rows[1], agg_sp.at[dstv[1]], ssem[1]).wait()
        plsc.subcore_barrier()

        @pl.when(c == 0)
        def _():
            pltpu.sync_copy(agg_sp.at[pl.ds(r0, rows_pt), :],
                            o0_hbm.at[pl.ds(r0, rows_pt), :])

        @pl.when(c == 1)
        def _():
            pltpu.sync_copy(agg_sp.at[pl.ds(r0, rows_pt), :],
                            o1_hbm.at[pl.ds(r0, rows_pt), :])

    return k


def _tc_mid(p0, p1, root1, bias1, W2t, root2, bias2, N, H, RL):
    Bn = 5000
    grid = (N // Bn,)

    def body(p0_ref, p1_ref, r1_ref, b1_ref, w2_ref, r2_ref, b2_ref,
             xw_ref, ob_ref):
        x = p0_ref[...] + p1_ref[...] + r1_ref[...] + b1_ref[...]
        x = jnp.maximum(x, 0.0)
        xw_ref[...] = jnp.dot(x, w2_ref[...], preferred_element_type=jnp.float32)
        ob_ref[...] = (jnp.dot(x, r2_ref[...], preferred_element_type=jnp.float32)
                       + b2_ref[...])

    row_spec = pl.BlockSpec((Bn, H), lambda i: (i, 0))
    full = lambda shp: pl.BlockSpec(shp, lambda i: (0, 0))
    return pl.pallas_call(
        body,
        grid=grid,
        in_specs=[row_spec, row_spec, row_spec, full((1, H)),
                  full((H, RL)), full((H, H)), full((1, H))],
        out_specs=[pl.BlockSpec((Bn, RL), lambda i: (i, 0)), row_spec],
        out_shape=(jax.ShapeDtypeStruct((N, RL), jnp.float32),
                   jax.ShapeDtypeStruct((N, H), jnp.float32)),
    )(p0, p1, root1, bias1, W2t, root2, bias2)


def _tc_final(q0, q1, ob, N, H):
    Bn = 5000
    grid = (N // Bn,)

    def body(q0_ref, q1_ref, ob_ref, o_ref):
        o_ref[...] = jax.nn.sigmoid(q0_ref[...] + q1_ref[...] + ob_ref[...])

    row_spec = pl.BlockSpec((Bn, H), lambda i: (i, 0))
    return pl.pallas_call(
        body,
        grid=grid,
        in_specs=[row_spec, row_spec, row_spec],
        out_specs=row_spec,
        out_shape=jax.ShapeDtypeStruct((N, H), jnp.float32),
    )(q0, q1, ob)


def kernel(edge_index, edge_type, W1, root1, bias1, W2, root2, bias2):
    R, N, H = W1.shape
    L = W2.shape[2]
    E = edge_index.shape[1]

    EW = NC * NS * CHUNK * 2
    Ep = ((E + EW - 1) // EW) * EW
    pad = Ep - E
    src = jnp.concatenate(
        [edge_index[0].astype(jnp.int32), jnp.zeros((pad,), jnp.int32)])
    dst = jnp.concatenate(
        [edge_index[1].astype(jnp.int32),
         N + (jnp.arange(pad, dtype=jnp.int32) % 64)])
    rel = jnp.concatenate(
        [edge_type.astype(jnp.int32), jnp.zeros((pad,), jnp.int32)])
    eb = jnp.stack([src.reshape(-1, CHUNK), dst.reshape(-1, CHUNK),
                    rel.reshape(-1, CHUNK)], axis=1)

    AGGR = ((N + 96) // NS + 7) // 8 * 8 * NS
    SEGN = ((N + 64) * R + NS * 64 - 1) // (NS * 64) * (NS * 64)

    zc = jnp.zeros((SEGN,), jnp.float32)
    ones = jnp.ones((CHUNK,), jnp.float32)
    zr = jnp.zeros((AGGR, H), jnp.float32)

    scales = _make_scales_kernel(Ep, SEGN, R)(eb, zc, ones)

    W1f = W1.reshape(R * N, H)
    p0, p1 = _make_agg_kernel(Ep, R * N, N, IREL, ISRC, AGGR, H)(
        eb, scales, W1f, zr)

    W2t = jnp.transpose(W2, (1, 0, 2)).reshape(H, R * L)
    xw, ob = _tc_mid(p0, p1, root1, bias1.reshape(1, H),
                     W2t, root2, bias2.reshape(1, L), N, H, R * L)

    q0, q1 = _make_agg_kernel(Ep, N * R, R, ISRC, IREL, AGGR, L)(
        eb, scales, xw.reshape(N * R, L), zr)

    return _tc_final(q0, q1, ob, N, L)


# speedup vs baseline: 48.4210x; 1.1010x over previous
import functools

import jax
import jax.numpy as jnp
from jax import lax
from jax.experimental import pallas as pl
from jax.experimental.pallas import tpu as pltpu
from jax.experimental.pallas import tpu_sc as plsc

NC = 2
NS = 16
CHUNK = 128
ISRC, IDST, IREL = 0, 1, 2


def _mesh():
    return plsc.VectorSubcoreMesh(core_axis_name="c", subcore_axis_name="s")


def _make_scales_kernel(Ep, SEGN, R, S=2):
    steps_all = Ep // (NS * CHUNK * S)
    steps_half = Ep // (NC * NS * CHUNK * S)
    per_tile = SEGN // NS
    BUFZ = per_tile // 8

    @functools.partial(
        pl.kernel,
        out_type=jax.ShapeDtypeStruct((Ep // CHUNK, CHUNK), jnp.float32),
        mesh=_mesh(),
        compiler_params=pltpu.CompilerParams(use_tc_tiling_on_sc=False),
        scratch_types=[
            pltpu.VMEM((S, 3, CHUNK), jnp.int32),
            pltpu.VMEM((S, 3, CHUNK), jnp.int32),
            pltpu.VMEM((S, CHUNK), jnp.int32),
            pltpu.VMEM((S, CHUNK), jnp.int32),
            pltpu.VMEM((S, CHUNK), jnp.float32),
            pltpu.VMEM((S, CHUNK), jnp.float32),
            pltpu.VMEM((CHUNK,), jnp.float32),
            pltpu.VMEM((BUFZ,), jnp.float32),
            pltpu.VMEM_SHARED((SEGN,), jnp.float32),
            pltpu.SemaphoreType.DMA,
            pltpu.SemaphoreType.DMA,
            pltpu.SemaphoreType.DMA,
            pltpu.SemaphoreType.DMA,
        ],
    )
    def k(eb_hbm, zc_hbm, ones_hbm, scales_hbm,
          eb0, eb1, segv0, segv1, sv0, sv1, fv, zbuf, cnt_sp,
          lin0, lin1, st0, st1):
        c = lax.axis_index("c")
        s = lax.axis_index("s")
        w = c * NS + s
        eb = (eb0, eb1)
        segv = (segv0, segv1)
        sv = (sv0, sv1)
        lin = (lin0, lin1)
        st = (st0, st1)

        def seg_all(b):
            for k2 in range(S):
                for i in range(CHUNK // 16):
                    sl = pl.ds(i * 16, 16)
                    segv[b][k2, sl] = (eb[b][k2, IDST, sl] * R
                                       + eb[b][k2, IREL, sl])

        pltpu.sync_copy(zc_hbm.at[pl.ds(s * per_tile, per_tile)],
                        cnt_sp.at[pl.ds(s * per_tile, per_tile)])
        pltpu.sync_copy(ones_hbm, fv)
        plsc.subcore_barrier()

        base1 = s * steps_all
        pltpu.async_copy(eb_hbm.at[pl.ds(base1 * S, S)], eb[0], lin[0])

        def p1(g, carry):
            for b in range(2):
                p, n = b, 1 - b
                cc = 2 * g + b
                @pl.when(cc + 1 < steps_all)
                def _():
                    pltpu.async_copy(eb_hbm.at[pl.ds((base1 + cc + 1) * S, S)],
                                     eb[n], lin[n])
                pltpu.make_async_copy(eb_hbm.at[pl.ds((base1 + cc) * S, S)],
                                      eb[p], lin[p]).wait()
                @pl.when(g >= 1)
                def _():
                    for k2 in range(S):
                        pltpu.make_async_copy(
                            fv, cnt_sp.at[segv[p].at[k2]], st[p]).wait()
                seg_all(p)
                for k2 in range(S):
                    pltpu.async_copy(fv, cnt_sp.at[segv[p].at[k2]], st[p],
                                     add=True)
            return carry
        lax.fori_loop(0, steps_all // 2, p1, 0)
        for b in range(2):
            for k2 in range(S):
                pltpu.make_async_copy(fv, cnt_sp.at[segv[b].at[k2]],
                                      st[b]).wait()
        plsc.subcore_barrier()

        for j in range(8):
            off = s * per_tile + j * BUFZ
            pltpu.sync_copy(cnt_sp.at[pl.ds(off, BUFZ)], zbuf)

            def inv(jj, carry):
                sl = pl.ds(jj * 16, 16)
                zbuf[sl] = 1.0 / jnp.maximum(zbuf[sl], 1.0)
                return carry
            lax.fori_loop(0, BUFZ // 16, inv, 0)
            pltpu.sync_copy(zbuf, cnt_sp.at[pl.ds(off, BUFZ)])
        plsc.subcore_barrier()

        base3 = w * steps_half
        pltpu.async_copy(eb_hbm.at[pl.ds(base3 * S, S)], eb[0], lin[0])

        def p3(g, carry):
            for b in range(2):
                p, n = b, 1 - b
                cc = 2 * g + b
                @pl.when(cc + 1 < steps_half)
                def _():
                    pltpu.async_copy(eb_hbm.at[pl.ds((base3 + cc + 1) * S, S)],
                                     eb[n], lin[n])
                pltpu.make_async_copy(eb_hbm.at[pl.ds((base3 + cc) * S, S)],
                                      eb[p], lin[p]).wait()
                @pl.when(g >= 1)
                def _():
                    pltpu.make_async_copy(
                        sv[p],
                        scales_hbm.at[pl.ds((base3 + cc - 2) * S, S), :],
                        st[p]).wait()
                seg_all(p)
                for k2 in range(S):
                    pltpu.sync_copy(cnt_sp.at[segv[p].at[k2]],
                                    sv[p].at[k2])
                pltpu.async_copy(
                    sv[p],
                    scales_hbm.at[pl.ds((base3 + cc) * S, S), :],
                    st[p])
            return carry
        lax.fori_loop(0, steps_half // 2, p3, 0)
        last = base3 + steps_half
        pltpu.make_async_copy(
            sv[0], scales_hbm.at[pl.ds((last - 2) * S, S), :],
            st[0]).wait()
        pltpu.make_async_copy(
            sv[1], scales_hbm.at[pl.ds((last - 1) * S, S), :],
            st[1]).wait()

    return k


def _make_agg_kernel(Ep, T, M, IA, IB, AGGR, H):
    chunks = Ep // (NC * NS * CHUNK)
    rows_pt = AGGR // NS

    @functools.partial(
        pl.kernel,
        out_type=(jax.ShapeDtypeStruct((AGGR, H), jnp.float32),
                  jax.ShapeDtypeStruct((AGGR, H), jnp.float32)),
        mesh=_mesh(),
        compiler_params=pltpu.CompilerParams(use_tc_tiling_on_sc=False),
        scratch_types=[
            pltpu.VMEM((3, CHUNK), jnp.int32),
            pltpu.VMEM((3, CHUNK), jnp.int32),
            pltpu.VMEM((CHUNK,), jnp.int32),
            pltpu.VMEM((CHUNK,), jnp.int32),
            pltpu.VMEM((CHUNK,), jnp.float32),
            pltpu.VMEM((CHUNK,), jnp.float32),
            pltpu.VMEM((CHUNK,), jnp.int32),
            pltpu.VMEM((CHUNK,), jnp.int32),
            pltpu.VMEM((CHUNK, H), jnp.float32),
            pltpu.VMEM((CHUNK, H), jnp.float32),
            pltpu.VMEM_SHARED((AGGR, H), jnp.float32),
            pltpu.SemaphoreType.DMA,
            pltpu.SemaphoreType.DMA,
            pltpu.SemaphoreType.DMA,
            pltpu.SemaphoreType.DMA,
            pltpu.SemaphoreType.DMA,
            pltpu.SemaphoreType.DMA,
        ],
    )
    def k(eb_hbm, sc_hbm, tab_hbm, zr_hbm, o0_hbm, o1_hbm,
          eb0, eb1, gidx0, gidx1, sv0, sv1, dstv0, dstv1, rows0, rows1,
          agg_sp, lin0, lin1, gs0, gs1, ss0, ss1):
        c = lax.axis_index("c")
        s = lax.axis_index("s")
        w = c * NS + s
        r0 = s * rows_pt
        eb = (eb0, eb1)
        gidx = (gidx0, gidx1)
        sv = (sv0, sv1)
        dstv = (dstv0, dstv1)
        rows = (rows0, rows1)
        lin = (lin0, lin1)
        gsem = (gs0, gs1)
        ssem = (ss0, ss1)

        pltpu.sync_copy(zr_hbm.at[pl.ds(r0, rows_pt), :],
                        agg_sp.at[pl.ds(r0, rows_pt), :])
        plsc.subcore_barrier()

        base = w * chunks

        def fire_lin(cc, b):
            pltpu.async_copy(eb_hbm.at[base + cc], eb[b], lin[b])
            pltpu.async_copy(sc_hbm.at[base + cc], sv[b], lin[b])

        def wait_lin(cc, b):
            pltpu.make_async_copy(eb_hbm.at[base + cc], eb[b], lin[b]).wait()
            pltpu.make_async_copy(sc_hbm.at[base + cc], sv[b], lin[b]).wait()

        def fire_gather(b):
            pltpu.async_copy(tab_hbm.at[gidx[b]], rows[b], gsem[b])

        fire_lin(0, 0)
        wait_lin(0, 0)
        for i in range(CHUNK // 16):
            sl = pl.ds(i * 16, 16)
            gidx[0][sl] = eb[0][IA, sl] * M + eb[0][IB, sl]
        fire_gather(0)

        def body(g, carry):
            for b in range(2):
                p, n = b, 1 - b
                cc = 2 * g + b
                @pl.when(cc + 1 < chunks)
                def _():
                    fire_lin(cc + 1, n)
                pltpu.make_async_copy(tab_hbm.at[gidx[p]], rows[p],
                                      gsem[p]).wait()
                for g8 in range(CHUNK // 16):
                    sl = pl.ds(g8 * 16, 16)
                    dstv[p][sl] = eb[p][IDST, sl]
                    s16 = sv[p][sl]
                    for ei in range(16):
                        e = g8 * 16 + ei
                        rows[p][e, :] = rows[p][e, :] * s16[ei]
                pltpu.async_copy(rows[p], agg_sp.at[dstv[p]], ssem[p],
                                 add=True)
                @pl.when(cc + 1 < chunks)
                def _():
                    wait_lin(cc + 1, n)
                    @pl.when(cc >= 1)
                    def _():
                        pltpu.make_async_copy(
                            rows[n], agg_sp.at[dstv[n]], ssem[n]).wait()
                    for i in range(CHUNK // 16):
                        sl = pl.ds(i * 16, 16)
                        gidx[n][sl] = (eb[n][IA, sl] * M + eb[n][IB, sl])
                    fire_gather(n)
            return carry
        lax.fori_loop(0, chunks // 2, body, 0)
        pltpu.make_async_copy(rows[0], agg_sp.at[dstv[0]], ssem[0]).wait()
        pltpu.make_async_copy(rows[1], agg_sp.at[dstv[1]], ssem[1]).wait()
        plsc.subcore_barrier()

        @pl.when(c == 0)
        def _():
            pltpu.sync_copy(agg_sp.at[pl.ds(r0, rows_pt), :],
                            o0_hbm.at[pl.ds(r0, rows_pt), :])

        @pl.when(c == 1)
        def _():
            pltpu.sync_copy(agg_sp.at[pl.ds(r0, rows_pt), :],
                            o1_hbm.at[pl.ds(r0, rows_pt), :])

    return k


def _tc_mid(p0, p1, root1, bias1, W2t, root2, bias2, N, H, RL):
    Bn = 5000
    grid = (N // Bn,)

    def body(p0_ref, p1_ref, r1_ref, b1_ref, w2_ref, r2_ref, b2_ref,
             xw_ref, ob_ref):
        x = p0_ref[...] + p1_ref[...] + r1_ref[...] + b1_ref[...]
        x = jnp.maximum(x, 0.0)
        xw_ref[...] = jnp.dot(x, w2_ref[...], preferred_element_type=jnp.float32)
        ob_ref[...] = (jnp.dot(x, r2_ref[...], preferred_element_type=jnp.float32)
                       + b2_ref[...])

    row_spec = pl.BlockSpec((Bn, H), lambda i: (i, 0))
    full = lambda shp: pl.BlockSpec(shp, lambda i: (0, 0))
    return pl.pallas_call(
        body,
        grid=grid,
        in_specs=[row_spec, row_spec, row_spec, full((1, H)),
                  full((H, RL)), full((H, H)), full((1, H))],
        out_specs=[pl.BlockSpec((Bn, RL), lambda i: (i, 0)), row_spec],
        out_shape=(jax.ShapeDtypeStruct((N, RL), jnp.float32),
                   jax.ShapeDtypeStruct((N, H), jnp.float32)),
    )(p0, p1, root1, bias1, W2t, root2, bias2)


def _tc_final(q0, q1, ob, N, H):
    Bn = 5000
    grid = (N // Bn,)

    def body(q0_ref, q1_ref, ob_ref, o_ref):
        o_ref[...] = jax.nn.sigmoid(q0_ref[...] + q1_ref[...] + ob_ref[...])

    row_spec = pl.BlockSpec((Bn, H), lambda i: (i, 0))
    return pl.pallas_call(
        body,
        grid=grid,
        in_specs=[row_spec, row_spec, row_spec],
        out_specs=row_spec,
        out_shape=jax.ShapeDtypeStruct((N, H), jnp.float32),
    )(q0, q1, ob)


def kernel(edge_index, edge_type, W1, root1, bias1, W2, root2, bias2):
    R, N, H = W1.shape
    L = W2.shape[2]
    E = edge_index.shape[1]

    EW = NC * NS * CHUNK * 4
    Ep = ((E + EW - 1) // EW) * EW
    pad = Ep - E
    src = jnp.concatenate(
        [edge_index[0].astype(jnp.int32), jnp.zeros((pad,), jnp.int32)])
    dst = jnp.concatenate(
        [edge_index[1].astype(jnp.int32),
         N + (jnp.arange(pad, dtype=jnp.int32) % 64)])
    rel = jnp.concatenate(
        [edge_type.astype(jnp.int32), jnp.zeros((pad,), jnp.int32)])
    eb = jnp.stack([src.reshape(-1, CHUNK), dst.reshape(-1, CHUNK),
                    rel.reshape(-1, CHUNK)], axis=1)

    AGGR = ((N + 96) // NS + 7) // 8 * 8 * NS
    SEGN = ((N + 64) * R + NS * 64 - 1) // (NS * 64) * (NS * 64)

    zc = jnp.zeros((SEGN,), jnp.float32)
    ones = jnp.ones((CHUNK,), jnp.float32)
    zr = jnp.zeros((AGGR, H), jnp.float32)

    scales = _make_scales_kernel(Ep, SEGN, R)(eb, zc, ones)

    W1f = W1.reshape(R * N, H)
    p0, p1 = _make_agg_kernel(Ep, R * N, N, IREL, ISRC, AGGR, H)(
        eb, scales, W1f, zr)

    W2t = jnp.transpose(W2, (1, 0, 2)).reshape(H, R * L)
    xw, ob = _tc_mid(p0, p1, root1, bias1.reshape(1, H),
                     W2t, root2, bias2.reshape(1, L), N, H, R * L)

    q0, q1 = _make_agg_kernel(Ep, N * R, R, ISRC, IREL, AGGR, L)(
        eb, scales, xw.reshape(N * R, L), zr)

    return _tc_final(q0, q1, ob, N, L)


# speedup vs baseline: 53.0861x; 1.0963x over previous
import functools

import jax
import jax.numpy as jnp
from jax import lax
from jax.experimental import pallas as pl
from jax.experimental.pallas import tpu as pltpu
from jax.experimental.pallas import tpu_sc as plsc

NC = 2
NS = 16
CHUNK = 128
ISRC, IDST, IREL = 0, 1, 2


def _mesh():
    return plsc.VectorSubcoreMesh(core_axis_name="c", subcore_axis_name="s")


def _make_scales_kernel(Ep, SEGN, R, S=2):
    steps_all = Ep // (NS * CHUNK * S)
    steps_half = Ep // (NC * NS * CHUNK * S)
    per_tile = SEGN // NS
    BUFZ = per_tile // 8

    @functools.partial(
        pl.kernel,
        out_type=jax.ShapeDtypeStruct((Ep // CHUNK, CHUNK), jnp.float32),
        mesh=_mesh(),
        compiler_params=pltpu.CompilerParams(use_tc_tiling_on_sc=False),
        scratch_types=[
            pltpu.VMEM((S, 3, CHUNK), jnp.int32),
            pltpu.VMEM((S, 3, CHUNK), jnp.int32),
            pltpu.VMEM((S, CHUNK), jnp.int32),
            pltpu.VMEM((S, CHUNK), jnp.int32),
            pltpu.VMEM((S, CHUNK), jnp.float32),
            pltpu.VMEM((S, CHUNK), jnp.float32),
            pltpu.VMEM((CHUNK,), jnp.float32),
            pltpu.VMEM((BUFZ,), jnp.float32),
            pltpu.VMEM_SHARED((SEGN,), jnp.float32),
            pltpu.SemaphoreType.DMA,
            pltpu.SemaphoreType.DMA,
            pltpu.SemaphoreType.DMA,
            pltpu.SemaphoreType.DMA,
        ],
    )
    def k(eb_hbm, scales_hbm,
          eb0, eb1, segv0, segv1, sv0, sv1, fv, zbuf, cnt_sp,
          lin0, lin1, st0, st1):
        c = lax.axis_index("c")
        s = lax.axis_index("s")
        w = c * NS + s
        eb = (eb0, eb1)
        segv = (segv0, segv1)
        sv = (sv0, sv1)
        lin = (lin0, lin1)
        st = (st0, st1)

        def seg_all(b):
            for k2 in range(S):
                for i in range(CHUNK // 16):
                    sl = pl.ds(i * 16, 16)
                    segv[b][k2, sl] = (eb[b][k2, IDST, sl] * R
                                       + eb[b][k2, IREL, sl])

        def zfill(jj, carry):
            zbuf[pl.ds(jj * 16, 16)] = jnp.zeros((16,), jnp.float32)
            return carry
        lax.fori_loop(0, BUFZ // 16, zfill, 0)
        for i in range(CHUNK // 16):
            fv[pl.ds(i * 16, 16)] = jnp.ones((16,), jnp.float32)
        for j in range(8):
            pltpu.sync_copy(zbuf,
                            cnt_sp.at[pl.ds(s * per_tile + j * BUFZ, BUFZ)])
        plsc.subcore_barrier()

        base1 = s * steps_all
        pltpu.async_copy(eb_hbm.at[pl.ds(base1 * S, S)], eb[0], lin[0])

        def p1(g, carry):
            for b in range(2):
                p, n = b, 1 - b
                cc = 2 * g + b
                @pl.when(cc + 1 < steps_all)
                def _():
                    pltpu.async_copy(eb_hbm.at[pl.ds((base1 + cc + 1) * S, S)],
                                     eb[n], lin[n])
                pltpu.make_async_copy(eb_hbm.at[pl.ds((base1 + cc) * S, S)],
                                      eb[p], lin[p]).wait()
                @pl.when(g >= 1)
                def _():
                    for k2 in range(S):
                        pltpu.make_async_copy(
                            fv, cnt_sp.at[segv[p].at[k2]], st[p]).wait()
                seg_all(p)
                for k2 in range(S):
                    pltpu.async_copy(fv, cnt_sp.at[segv[p].at[k2]], st[p],
                                     add=True)
            return carry
        lax.fori_loop(0, steps_all // 2, p1, 0)
        for b in range(2):
            for k2 in range(S):
                pltpu.make_async_copy(fv, cnt_sp.at[segv[b].at[k2]],
                                      st[b]).wait()
        plsc.subcore_barrier()

        for j in range(8):
            off = s * per_tile + j * BUFZ
            pltpu.sync_copy(cnt_sp.at[pl.ds(off, BUFZ)], zbuf)

            def inv(jj, carry):
                sl = pl.ds(jj * 16, 16)
                zbuf[sl] = 1.0 / jnp.maximum(zbuf[sl], 1.0)
                return carry
            lax.fori_loop(0, BUFZ // 16, inv, 0)
            pltpu.sync_copy(zbuf, cnt_sp.at[pl.ds(off, BUFZ)])
        plsc.subcore_barrier()

        base3 = w * steps_half
        pltpu.async_copy(eb_hbm.at[pl.ds(base3 * S, S)], eb[0], lin[0])

        def p3(g, carry):
            for b in range(2):
                p, n = b, 1 - b
                cc = 2 * g + b
                @pl.when(cc + 1 < steps_half)
                def _():
                    pltpu.async_copy(eb_hbm.at[pl.ds((base3 + cc + 1) * S, S)],
                                     eb[n], lin[n])
                pltpu.make_async_copy(eb_hbm.at[pl.ds((base3 + cc) * S, S)],
                                      eb[p], lin[p]).wait()
                @pl.when(g >= 1)
                def _():
                    pltpu.make_async_copy(
                        sv[p],
                        scales_hbm.at[pl.ds((base3 + cc - 2) * S, S), :],
                        st[p]).wait()
                seg_all(p)
                for k2 in range(S):
                    pltpu.sync_copy(cnt_sp.at[segv[p].at[k2]],
                                    sv[p].at[k2])
                pltpu.async_copy(
                    sv[p],
                    scales_hbm.at[pl.ds((base3 + cc) * S, S), :],
                    st[p])
            return carry
        lax.fori_loop(0, steps_half // 2, p3, 0)
        last = base3 + steps_half
        pltpu.make_async_copy(
            sv[0], scales_hbm.at[pl.ds((last - 2) * S, S), :],
            st[0]).wait()
        pltpu.make_async_copy(
            sv[1], scales_hbm.at[pl.ds((last - 1) * S, S), :],
            st[1]).wait()

    return k


def _make_agg_kernel(Ep, T, M, IA, IB, AGGR, H):
    chunks = Ep // (NC * NS * CHUNK)
    rows_pt = AGGR // NS

    @functools.partial(
        pl.kernel,
        out_type=(jax.ShapeDtypeStruct((AGGR, H), jnp.float32),
                  jax.ShapeDtypeStruct((AGGR, H), jnp.float32)),
        mesh=_mesh(),
        compiler_params=pltpu.CompilerParams(use_tc_tiling_on_sc=False),
        scratch_types=[
            pltpu.VMEM((3, CHUNK), jnp.int32),
            pltpu.VMEM((3, CHUNK), jnp.int32),
            pltpu.VMEM((CHUNK,), jnp.int32),
            pltpu.VMEM((CHUNK,), jnp.int32),
            pltpu.VMEM((CHUNK,), jnp.float32),
            pltpu.VMEM((CHUNK,), jnp.float32),
            pltpu.VMEM((CHUNK,), jnp.int32),
            pltpu.VMEM((CHUNK,), jnp.int32),
            pltpu.VMEM((CHUNK, H), jnp.float32),
            pltpu.VMEM((CHUNK, H), jnp.float32),
            pltpu.VMEM_SHARED((AGGR, H), jnp.float32),
            pltpu.SemaphoreType.DMA,
            pltpu.SemaphoreType.DMA,
            pltpu.SemaphoreType.DMA,
            pltpu.SemaphoreType.DMA,
            pltpu.SemaphoreType.DMA,
            pltpu.SemaphoreType.DMA,
        ],
    )
    def k(eb_hbm, sc_hbm, tab_hbm, o0_hbm, o1_hbm,
          eb0, eb1, gidx0, gidx1, sv0, sv1, dstv0, dstv1, rows0, rows1,
          agg_sp, lin0, lin1, gs0, gs1, ss0, ss1):
        c = lax.axis_index("c")
        s = lax.axis_index("s")
        w = c * NS + s
        r0 = s * rows_pt
        eb = (eb0, eb1)
        gidx = (gidx0, gidx1)
        sv = (sv0, sv1)
        dstv = (dstv0, dstv1)
        rows = (rows0, rows1)
        lin = (lin0, lin1)
        gsem = (gs0, gs1)
        ssem = (ss0, ss1)

        for e in range(CHUNK):
            rows0[e, :] = jnp.zeros((16,), jnp.float32)
        nfull, tail = rows_pt // CHUNK, rows_pt % CHUNK
        for j in range(nfull):
            pltpu.sync_copy(rows0,
                            agg_sp.at[pl.ds(r0 + j * CHUNK, CHUNK), :])
        if tail:
            pltpu.sync_copy(rows0.at[pl.ds(0, tail), :],
                            agg_sp.at[pl.ds(r0 + nfull * CHUNK, tail), :])
        plsc.subcore_barrier()

        base = w * chunks

        def fire_lin(cc, b):
            pltpu.async_copy(eb_hbm.at[base + cc], eb[b], lin[b])
            pltpu.async_copy(sc_hbm.at[base + cc], sv[b], lin[b])

        def wait_lin(cc, b):
            pltpu.make_async_copy(eb_hbm.at[base + cc], eb[b], lin[b]).wait()
            pltpu.make_async_copy(sc_hbm.at[base + cc], sv[b], lin[b]).wait()

        def fire_gather(b):
            pltpu.async_copy(tab_hbm.at[gidx[b]], rows[b], gsem[b])

        fire_lin(0, 0)
        wait_lin(0, 0)
        for i in range(CHUNK // 16):
            sl = pl.ds(i * 16, 16)
            gidx[0][sl] = eb[0][IA, sl] * M + eb[0][IB, sl]
        fire_gather(0)

        def body(g, carry):
            for b in range(2):
                p, n = b, 1 - b
                cc = 2 * g + b
                @pl.when(cc + 1 < chunks)
                def _():
                    fire_lin(cc + 1, n)
                pltpu.make_async_copy(tab_hbm.at[gidx[p]], rows[p],
                                      gsem[p]).wait()
                for g8 in range(CHUNK // 16):
                    sl = pl.ds(g8 * 16, 16)
                    dstv[p][sl] = eb[p][IDST, sl]
                    s16 = sv[p][sl]
                    for ei in range(16):
                        e = g8 * 16 + ei
                        rows[p][e, :] = rows[p][e, :] * s16[ei]
                pltpu.async_copy(rows[p], agg_sp.at[dstv[p]], ssem[p],
                                 add=True)
                @pl.when(cc + 1 < chunks)
                def _():
                    wait_lin(cc + 1, n)
                    @pl.when(cc >= 1)
                    def _():
                        pltpu.make_async_copy(
                            rows[n], agg_sp.at[dstv[n]], ssem[n]).wait()
                    for i in range(CHUNK // 16):
                        sl = pl.ds(i * 16, 16)
                        gidx[n][sl] = (eb[n][IA, sl] * M + eb[n][IB, sl])
                    fire_gather(n)
            return carry
        lax.fori_loop(0, chunks // 2, body, 0)
        pltpu.make_async_copy(rows[0], agg_sp.at[dstv[0]], ssem[0]).wait()
        pltpu.make_async_copy(rows[1], agg_sp.at[dstv[1]], ssem[1]).wait()
        plsc.subcore_barrier()

        @pl.when(c == 0)
        def _():
            pltpu.sync_copy(agg_sp.at[pl.ds(r0, rows_pt), :],
                            o0_hbm.at[pl.ds(r0, rows_pt), :])

        @pl.when(c == 1)
        def _():
            pltpu.sync_copy(agg_sp.at[pl.ds(r0, rows_pt), :],
                            o1_hbm.at[pl.ds(r0, rows_pt), :])

    return k


def _tc_mid(p0r, p1r, root1r, b1t, W2t, r2bd, b2t, NW8, H):
    Bm = 512
    grid = ((NW8 + Bm - 1) // Bm,)
    PACK = 128 // H

    def body(p0_ref, p1_ref, r1_ref, b1_ref, w2_ref, bd_ref, b2_ref,
             xw_ref, ob_ref):
        x8 = p0_ref[...] + p1_ref[...] + r1_ref[...] + b1_ref[...]
        x8 = jnp.maximum(x8, 0.0)
        for j in range(PACK):
            xj = x8[:, j * H:(j + 1) * H]
            xw_ref[:, j * 128:(j + 1) * 128] = jnp.dot(
                xj, w2_ref[...], preferred_element_type=jnp.float32)
        ob_ref[...] = (jnp.dot(x8, bd_ref[...],
                               preferred_element_type=jnp.float32)
                       + b2_ref[...])

    row_spec = pl.BlockSpec((Bm, 128), lambda i: (i, 0))
    full = lambda shp: pl.BlockSpec(shp, lambda i: (0, 0))
    return pl.pallas_call(
        body,
        grid=grid,
        in_specs=[row_spec, row_spec, row_spec, full((1, 128)),
                  full((H, 128)), full((128, 128)), full((1, 128))],
        out_specs=[pl.BlockSpec((Bm, PACK * 128), lambda i: (i, 0)), row_spec],
        out_shape=(jax.ShapeDtypeStruct((NW8, PACK * 128), jnp.float32),
                   jax.ShapeDtypeStruct((NW8, 128), jnp.float32)),
    )(p0r, p1r, root1r, b1t, W2t, r2bd, b2t)


def _tc_final(q0r, q1r, ob, NW8):
    Bm = 512
    grid = ((NW8 + Bm - 1) // Bm,)

    def body(q0_ref, q1_ref, ob_ref, o_ref):
        o_ref[...] = jax.nn.sigmoid(q0_ref[...] + q1_ref[...] + ob_ref[...])

    row_spec = pl.BlockSpec((Bm, 128), lambda i: (i, 0))
    return pl.pallas_call(
        body,
        grid=grid,
        in_specs=[row_spec, row_spec, row_spec],
        out_specs=row_spec,
        out_shape=jax.ShapeDtypeStruct((NW8, 128), jnp.float32),
    )(q0r, q1r, ob)


def kernel(edge_index, edge_type, W1, root1, bias1, W2, root2, bias2):
    R, N, H = W1.shape
    L = W2.shape[2]
    E = edge_index.shape[1]

    EW = NC * NS * CHUNK * 4
    Ep = ((E + EW - 1) // EW) * EW
    pad = Ep - E
    src = jnp.concatenate(
        [edge_index[0].astype(jnp.int32), jnp.zeros((pad,), jnp.int32)])
    dst = jnp.concatenate(
        [edge_index[1].astype(jnp.int32),
         N + (jnp.arange(pad, dtype=jnp.int32) % 64)])
    rel = jnp.concatenate(
        [edge_type.astype(jnp.int32), jnp.zeros((pad,), jnp.int32)])
    eb = jnp.stack([src.reshape(-1, CHUNK), dst.reshape(-1, CHUNK),
                    rel.reshape(-1, CHUNK)], axis=1)

    AGGR = ((N + 96) // NS + 7) // 8 * 8 * NS
    SEGN = ((N + 64) * R + NS * 64 - 1) // (NS * 64) * (NS * 64)

    scales = _make_scales_kernel(Ep, SEGN, R)(eb)

    W1f = W1.reshape(R * N, H)
    p0, p1 = _make_agg_kernel(Ep, R * N, N, IREL, ISRC, AGGR, H)(
        eb, scales, W1f)

    PACK = 128 // H
    NW8 = N // PACK
    AW8 = AGGR * H // 128
    W2t = jnp.transpose(W2, (1, 0, 2)).reshape(H, R * L)
    r2bd = jnp.kron(jnp.eye(PACK, dtype=jnp.float32), root2)
    b1t = jnp.tile(bias1, PACK).reshape(1, 128)
    b2t = jnp.tile(bias2, PACK).reshape(1, 128)
    root1r = root1.reshape(NW8, 128)

    xw, ob = _tc_mid(p0.reshape(AW8, 128), p1.reshape(AW8, 128),
                     root1r, b1t, W2t, r2bd, b2t, NW8, H)

    q0, q1 = _make_agg_kernel(Ep, N * R, R, ISRC, IREL, AGGR, L)(
        eb, scales, xw.reshape(N * R, L))

    out = _tc_final(q0.reshape(AW8, 128), q1.reshape(AW8, 128),
                    ob, NW8)
    return out.reshape(N, H)


# speedup vs baseline: 55.2861x; 1.0414x over previous
import functools

import jax
import jax.numpy as jnp
from jax import lax
from jax.experimental import pallas as pl
from jax.experimental.pallas import tpu as pltpu
from jax.experimental.pallas import tpu_sc as plsc

NC = 2
NS = 16
CHUNK = 128
ISRC, IDST, IREL = 0, 1, 2


def _mesh():
    return plsc.VectorSubcoreMesh(core_axis_name="c", subcore_axis_name="s")


def _make_scales_kernel(Ep, SEGN, R, S=4):
    steps_all = Ep // (NS * CHUNK * S)
    steps_half = Ep // (NC * NS * CHUNK * S)
    per_tile = SEGN // NS
    BUFZ = per_tile // 8

    @functools.partial(
        pl.kernel,
        out_type=jax.ShapeDtypeStruct((Ep // CHUNK, CHUNK), jnp.float32),
        mesh=_mesh(),
        compiler_params=pltpu.CompilerParams(use_tc_tiling_on_sc=False),
        scratch_types=[
            pltpu.VMEM((S * 3 * CHUNK,), jnp.int32),
            pltpu.VMEM((S * 3 * CHUNK,), jnp.int32),
            pltpu.VMEM((S, CHUNK), jnp.int32),
            pltpu.VMEM((S, CHUNK), jnp.int32),
            pltpu.VMEM((S, CHUNK), jnp.float32),
            pltpu.VMEM((S, CHUNK), jnp.float32),
            pltpu.VMEM((CHUNK,), jnp.float32),
            pltpu.VMEM((BUFZ,), jnp.float32),
            pltpu.VMEM_SHARED((SEGN,), jnp.float32),
            pltpu.SemaphoreType.DMA,
            pltpu.SemaphoreType.DMA,
            pltpu.SemaphoreType.DMA,
            pltpu.SemaphoreType.DMA,
        ],
    )
    def k(eb_hbm, scales_hbm,
          eb0, eb1, segv0, segv1, sv0, sv1, fv, zbuf, cnt_sp,
          lin0, lin1, st0, st1):
        c = lax.axis_index("c")
        s = lax.axis_index("s")
        w = c * NS + s
        eb = (eb0, eb1)
        segv = (segv0, segv1)
        sv = (sv0, sv1)
        lin = (lin0, lin1)
        st = (st0, st1)

        def seg_all(b):
            for k2 in range(S):
                for i in range(CHUNK // 16):
                    sl = pl.ds(i * 16, 16)
                    dsl = pl.ds(k2 * 3 * CHUNK + IDST * CHUNK + i * 16, 16)
                    rsl = pl.ds(k2 * 3 * CHUNK + IREL * CHUNK + i * 16, 16)
                    segv[b][k2, sl] = eb[b][dsl] * R + eb[b][rsl]

        def zfill(jj, carry):
            zbuf[pl.ds(jj * 16, 16)] = jnp.zeros((16,), jnp.float32)
            return carry
        lax.fori_loop(0, BUFZ // 16, zfill, 0)
        for i in range(CHUNK // 16):
            fv[pl.ds(i * 16, 16)] = jnp.ones((16,), jnp.float32)
        for j in range(8):
            pltpu.sync_copy(zbuf,
                            cnt_sp.at[pl.ds(s * per_tile + j * BUFZ, BUFZ)])
        plsc.subcore_barrier()

        base1 = s * steps_all
        pltpu.async_copy(eb_hbm.at[pl.ds(base1 * S * 3 * CHUNK, S * 3 * CHUNK)], eb[0], lin[0])

        def p1(g, carry):
            for b in range(2):
                p, n = b, 1 - b
                cc = 2 * g + b
                @pl.when(cc + 1 < steps_all)
                def _():
                    pltpu.async_copy(
                        eb_hbm.at[pl.ds((base1 + cc + 1) * S * 3 * CHUNK,
                                        S * 3 * CHUNK)], eb[n], lin[n])
                pltpu.make_async_copy(
                    eb_hbm.at[pl.ds((base1 + cc) * S * 3 * CHUNK,
                                    S * 3 * CHUNK)], eb[p], lin[p]).wait()
                @pl.when(g >= 1)
                def _():
                    for k2 in range(S):
                        pltpu.make_async_copy(
                            fv, cnt_sp.at[segv[p].at[k2]], st[p]).wait()
                seg_all(p)
                for k2 in range(S):
                    pltpu.async_copy(fv, cnt_sp.at[segv[p].at[k2]], st[p],
                                     add=True)
            return carry
        lax.fori_loop(0, steps_all // 2, p1, 0)
        for b in range(2):
            for k2 in range(S):
                pltpu.make_async_copy(fv, cnt_sp.at[segv[b].at[k2]],
                                      st[b]).wait()
        plsc.subcore_barrier()

        for j in range(8):
            off = s * per_tile + j * BUFZ
            pltpu.sync_copy(cnt_sp.at[pl.ds(off, BUFZ)], zbuf)

            def inv(jj, carry):
                sl = pl.ds(jj * 16, 16)
                zbuf[sl] = 1.0 / jnp.maximum(zbuf[sl], 1.0)
                return carry
            lax.fori_loop(0, BUFZ // 16, inv, 0)
            pltpu.sync_copy(zbuf, cnt_sp.at[pl.ds(off, BUFZ)])
        plsc.subcore_barrier()

        base3 = w * steps_half
        pltpu.async_copy(eb_hbm.at[pl.ds(base3 * S * 3 * CHUNK, S * 3 * CHUNK)], eb[0], lin[0])

        def p3(g, carry):
            for b in range(2):
                p, n = b, 1 - b
                cc = 2 * g + b
                @pl.when(cc + 1 < steps_half)
                def _():
                    pltpu.async_copy(
                        eb_hbm.at[pl.ds((base3 + cc + 1) * S * 3 * CHUNK,
                                        S * 3 * CHUNK)], eb[n], lin[n])
                pltpu.make_async_copy(
                    eb_hbm.at[pl.ds((base3 + cc) * S * 3 * CHUNK,
                                    S * 3 * CHUNK)], eb[p], lin[p]).wait()
                @pl.when(g >= 1)
                def _():
                    pltpu.make_async_copy(
                        sv[p],
                        scales_hbm.at[pl.ds((base3 + cc - 2) * S, S), :],
                        st[p]).wait()
                seg_all(p)
                for k2 in range(S):
                    pltpu.sync_copy(cnt_sp.at[segv[p].at[k2]],
                                    sv[p].at[k2])
                pltpu.async_copy(
                    sv[p],
                    scales_hbm.at[pl.ds((base3 + cc) * S, S), :],
                    st[p])
            return carry
        lax.fori_loop(0, steps_half // 2, p3, 0)
        last = base3 + steps_half
        pltpu.make_async_copy(
            sv[0], scales_hbm.at[pl.ds((last - 2) * S, S), :],
            st[0]).wait()
        pltpu.make_async_copy(
            sv[1], scales_hbm.at[pl.ds((last - 1) * S, S), :],
            st[1]).wait()

    return k


def _make_agg_kernel(Ep, T, M, IA, IB, AGGR, H):
    chunks = Ep // (NC * NS * CHUNK)
    rows_pt = AGGR // NS

    @functools.partial(
        pl.kernel,
        out_type=(jax.ShapeDtypeStruct((AGGR, H), jnp.float32),
                  jax.ShapeDtypeStruct((AGGR, H), jnp.float32)),
        mesh=_mesh(),
        compiler_params=pltpu.CompilerParams(use_tc_tiling_on_sc=False),
        scratch_types=[
            pltpu.VMEM((3 * CHUNK,), jnp.int32),
            pltpu.VMEM((3 * CHUNK,), jnp.int32),
            pltpu.VMEM((CHUNK,), jnp.int32),
            pltpu.VMEM((CHUNK,), jnp.int32),
            pltpu.VMEM((CHUNK,), jnp.float32),
            pltpu.VMEM((CHUNK,), jnp.float32),
            pltpu.VMEM((CHUNK,), jnp.int32),
            pltpu.VMEM((CHUNK,), jnp.int32),
            pltpu.VMEM((CHUNK, H), jnp.float32),
            pltpu.VMEM((CHUNK, H), jnp.float32),
            pltpu.VMEM_SHARED((AGGR, H), jnp.float32),
            pltpu.SemaphoreType.DMA,
            pltpu.SemaphoreType.DMA,
            pltpu.SemaphoreType.DMA,
            pltpu.SemaphoreType.DMA,
            pltpu.SemaphoreType.DMA,
            pltpu.SemaphoreType.DMA,
        ],
    )
    def k(eb_hbm, sc_hbm, tab_hbm, o0_hbm, o1_hbm,
          eb0, eb1, gidx0, gidx1, sv0, sv1, dstv0, dstv1, rows0, rows1,
          agg_sp, lin0, lin1, gs0, gs1, ss0, ss1):
        c = lax.axis_index("c")
        s = lax.axis_index("s")
        w = c * NS + s
        r0 = s * rows_pt
        eb = (eb0, eb1)
        gidx = (gidx0, gidx1)
        sv = (sv0, sv1)
        dstv = (dstv0, dstv1)
        rows = (rows0, rows1)
        lin = (lin0, lin1)
        gsem = (gs0, gs1)
        ssem = (ss0, ss1)

        for e in range(CHUNK):
            rows0[e, :] = jnp.zeros((16,), jnp.float32)
        nfull, tail = rows_pt // CHUNK, rows_pt % CHUNK
        for j in range(nfull):
            pltpu.sync_copy(rows0,
                            agg_sp.at[pl.ds(r0 + j * CHUNK, CHUNK), :])
        if tail:
            pltpu.sync_copy(rows0.at[pl.ds(0, tail), :],
                            agg_sp.at[pl.ds(r0 + nfull * CHUNK, tail), :])
        plsc.subcore_barrier()

        base = w * chunks

        def fire_lin(cc, b):
            pltpu.async_copy(eb_hbm.at[pl.ds((base + cc) * 3 * CHUNK,
                                             3 * CHUNK)], eb[b], lin[b])
            pltpu.async_copy(sc_hbm.at[base + cc], sv[b], lin[b])

        def wait_lin(cc, b):
            pltpu.make_async_copy(eb_hbm.at[pl.ds((base + cc) * 3 * CHUNK,
                                                  3 * CHUNK)], eb[b],
                                  lin[b]).wait()
            pltpu.make_async_copy(sc_hbm.at[base + cc], sv[b], lin[b]).wait()

        def fire_gather(b):
            pltpu.async_copy(tab_hbm.at[gidx[b]], rows[b], gsem[b])

        fire_lin(0, 0)
        wait_lin(0, 0)
        for i in range(CHUNK // 16):
            sl = pl.ds(i * 16, 16)
            gidx[0][sl] = (eb[0][pl.ds(IA * CHUNK + i * 16, 16)] * M
                           + eb[0][pl.ds(IB * CHUNK + i * 16, 16)])
        fire_gather(0)

        def body(g, carry):
            for b in range(2):
                p, n = b, 1 - b
                cc = 2 * g + b
                @pl.when(cc + 1 < chunks)
                def _():
                    fire_lin(cc + 1, n)
                pltpu.make_async_copy(tab_hbm.at[gidx[p]], rows[p],
                                      gsem[p]).wait()
                for g8 in range(CHUNK // 16):
                    sl = pl.ds(g8 * 16, 16)
                    dstv[p][sl] = eb[p][pl.ds(IDST * CHUNK + g8 * 16, 16)]
                    s16 = sv[p][sl]
                    for ei in range(16):
                        e = g8 * 16 + ei
                        rows[p][e, :] = rows[p][e, :] * s16[ei]
                pltpu.async_copy(rows[p], agg_sp.at[dstv[p]], ssem[p],
                                 add=True)
                @pl.when(cc + 1 < chunks)
                def _():
                    wait_lin(cc + 1, n)
                    @pl.when(cc >= 1)
                    def _():
                        pltpu.make_async_copy(
                            rows[n], agg_sp.at[dstv[n]], ssem[n]).wait()
                    for i in range(CHUNK // 16):
                        sl = pl.ds(i * 16, 16)
                        gidx[n][sl] = (
                            eb[n][pl.ds(IA * CHUNK + i * 16, 16)] * M
                            + eb[n][pl.ds(IB * CHUNK + i * 16, 16)])
                    fire_gather(n)
            return carry
        lax.fori_loop(0, chunks // 2, body, 0)
        pltpu.make_async_copy(rows[0], agg_sp.at[dstv[0]], ssem[0]).wait()
        pltpu.make_async_copy(rows[1], agg_sp.at[dstv[1]], ssem[1]).wait()
        plsc.subcore_barrier()

        @pl.when(c == 0)
        def _():
            pltpu.sync_copy(agg_sp.at[pl.ds(r0, rows_pt), :],
                            o0_hbm.at[pl.ds(r0, rows_pt), :])

        @pl.when(c == 1)
        def _():
            pltpu.sync_copy(agg_sp.at[pl.ds(r0, rows_pt), :],
                            o1_hbm.at[pl.ds(r0, rows_pt), :])

    return k


def _tc_mid(p0r, p1r, root1r, b1t, W2t, r2bd, b2t, NW8, H):
    Bm = 512
    grid = ((NW8 + Bm - 1) // Bm,)
    PACK = 128 // H

    def body(p0_ref, p1_ref, r1_ref, b1_ref, w2_ref, bd_ref, b2_ref,
             xw_ref, ob_ref):
        x8 = p0_ref[...] + p1_ref[...] + r1_ref[...] + b1_ref[...]
        x8 = jnp.maximum(x8, 0.0)
        for j in range(PACK):
            xj = x8[:, j * H:(j + 1) * H]
            xw_ref[:, j * 128:(j + 1) * 128] = jnp.dot(
                xj, w2_ref[...], preferred_element_type=jnp.float32)
        ob_ref[...] = (jnp.dot(x8, bd_ref[...],
                               preferred_element_type=jnp.float32)
                       + b2_ref[...])

    row_spec = pl.BlockSpec((Bm, 128), lambda i: (i, 0))
    full = lambda shp: pl.BlockSpec(shp, lambda i: (0, 0))
    return pl.pallas_call(
        body,
        grid=grid,
        in_specs=[row_spec, row_spec, row_spec, full((1, 128)),
                  full((H, 128)), full((128, 128)), full((1, 128))],
        out_specs=[pl.BlockSpec((Bm, PACK * 128), lambda i: (i, 0)), row_spec],
        out_shape=(jax.ShapeDtypeStruct((NW8, PACK * 128), jnp.float32),
                   jax.ShapeDtypeStruct((NW8, 128), jnp.float32)),
    )(p0r, p1r, root1r, b1t, W2t, r2bd, b2t)


def _tc_final(q0r, q1r, ob, NW8):
    Bm = 512
    grid = ((NW8 + Bm - 1) // Bm,)

    def body(q0_ref, q1_ref, ob_ref, o_ref):
        o_ref[...] = jax.nn.sigmoid(q0_ref[...] + q1_ref[...] + ob_ref[...])

    row_spec = pl.BlockSpec((Bm, 128), lambda i: (i, 0))
    return pl.pallas_call(
        body,
        grid=grid,
        in_specs=[row_spec, row_spec, row_spec],
        out_specs=row_spec,
        out_shape=jax.ShapeDtypeStruct((NW8, 128), jnp.float32),
    )(q0r, q1r, ob)


def kernel(edge_index, edge_type, W1, root1, bias1, W2, root2, bias2):
    R, N, H = W1.shape
    L = W2.shape[2]
    E = edge_index.shape[1]

    EW = NC * NS * CHUNK * 8
    Ep = ((E + EW - 1) // EW) * EW
    pad = Ep - E
    src = jnp.concatenate(
        [edge_index[0].astype(jnp.int32), jnp.zeros((pad,), jnp.int32)])
    dst = jnp.concatenate(
        [edge_index[1].astype(jnp.int32),
         N + (jnp.arange(pad, dtype=jnp.int32) % 64)])
    rel = jnp.concatenate(
        [edge_type.astype(jnp.int32), jnp.zeros((pad,), jnp.int32)])
    eb = jnp.stack([src.reshape(-1, CHUNK), dst.reshape(-1, CHUNK),
                    rel.reshape(-1, CHUNK)], axis=1).reshape(-1)

    AGGR = ((N + 96) // NS + 7) // 8 * 8 * NS
    SEGN = ((N + 64) * R + NS * 64 - 1) // (NS * 64) * (NS * 64)

    scales = _make_scales_kernel(Ep, SEGN, R)(eb)

    W1f = W1.reshape(R * N, H)
    p0, p1 = _make_agg_kernel(Ep, R * N, N, IREL, ISRC, AGGR, H)(
        eb, scales, W1f)

    PACK = 128 // H
    NW8 = N // PACK
    AW8 = AGGR * H // 128
    W2t = jnp.transpose(W2, (1, 0, 2)).reshape(H, R * L)
    r2bd = jnp.kron(jnp.eye(PACK, dtype=jnp.float32), root2)
    b1t = jnp.tile(bias1, PACK).reshape(1, 128)
    b2t = jnp.tile(bias2, PACK).reshape(1, 128)
    root1r = root1.reshape(NW8, 128)

    xw, ob = _tc_mid(p0.reshape(AW8, 128), p1.reshape(AW8, 128),
                     root1r, b1t, W2t, r2bd, b2t, NW8, H)

    q0, q1 = _make_agg_kernel(Ep, N * R, R, ISRC, IREL, AGGR, L)(
        eb, scales, xw.reshape(N * R, L))

    out = _tc_final(q0.reshape(AW8, 128), q1.reshape(AW8, 128),
                    ob, NW8)
    return out.reshape(N, H)
